# baseline (speedup 1.0000x reference)
import functools

import jax
import jax.numpy as jnp
from jax import lax
from jax.experimental import pallas as pl
from jax.experimental.pallas import tpu as pltpu
from jax.experimental.pallas import tpu_sc as plsc

NEG = -3.0e38
N_TOK = 4096
K_NBR = 9
RB = 256

def _sc_workers():
    info = plsc.get_sparse_core_info()
    return info.num_cores, info.num_subcores


def _pixel_shuffle(x, r):
    B, C, H, W = x.shape
    x = x.reshape(B, C // (r * r), r, r, H, W)
    x = x.transpose(0, 1, 4, 2, 5, 3)
    return x.reshape(B, C // (r * r), H * r, W * r)


def _conv1_body(xp_ref, w_ref, b_ref, o_ref):
    acc = jnp.zeros((16, 128 * 128), jnp.float32)
    for dy in range(3):
        for dx in range(3):
            xs = xp_ref[0, :, dy:dy + 128, dx:dx + 128].reshape(3, 128 * 128)
            acc = acc + lax.dot_general(
                w_ref[:, :, dy, dx], xs, (((1,), (0,)), ((), ())),
                preferred_element_type=jnp.float32)
    acc = jnp.maximum(acc + b_ref[:], 0.0)
    o_ref[0] = acc.reshape(16, 128, 128)


def _conv1(x, W1, b1):
    B = x.shape[0]
    xp = jnp.pad(x, ((0, 0), (0, 0), (1, 1), (1, 1)))
    return pl.pallas_call(
        _conv1_body,
        grid=(B,),
        in_specs=[
            pl.BlockSpec((1, 3, 130, 130), lambda b: (b, 0, 0, 0)),
            pl.BlockSpec((16, 3, 3, 3), lambda b: (0, 0, 0, 0)),
            pl.BlockSpec((16, 1), lambda b: (0, 0)),
        ],
        out_specs=pl.BlockSpec((1, 16, 128, 128), lambda b: (b, 0, 0, 0)),
        out_shape=jax.ShapeDtypeStruct((B, 16, 128, 128), jnp.float32),
    )(xp, W1, b1.reshape(16, 1))


def _topk_body(xf_ref, xr_ref, o_ref):
    x = xf_ref[0]
    N = x.shape[0]
    nrm = jnp.sqrt(jnp.sum(x * x, axis=1, keepdims=True))
    xn = x * (1.0 / (nrm + 1e-8))
    xr = xr_ref[0]
    rnrm = jnp.sqrt(jnp.sum(xr * xr, axis=1, keepdims=True))
    rows = xr * (1.0 / (rnrm + 1e-8))
    sim = lax.dot_general(rows, xn, (((1,), (1,)), ((), ())),
                          preferred_element_type=jnp.float32)
    iota = lax.broadcasted_iota(jnp.int32, sim.shape, 1)
    base = pl.program_id(0) * N
    for k in range(K_NBR):
        mx = jnp.max(sim, axis=1, keepdims=True)
        mi = jnp.where(sim == mx, iota, N)
        idxk = jnp.min(mi, axis=1, keepdims=True)
        o_ref[0, :, k] = (idxk + base)[:, 0]
        sim = jnp.where(mi == idxk, NEG, sim)


def _topk(xf):
    B, N, C = xf.shape
    return pl.pallas_call(
        _topk_body,
        grid=(B, N // RB),
        in_specs=[
            pl.BlockSpec((1, N, C), lambda b, i: (b, 0, 0)),
            pl.BlockSpec((1, RB, C), lambda b, i: (b, i, 0)),
        ],
        out_specs=pl.BlockSpec((1, RB, K_NBR), lambda b, i: (b, i, 0)),
        out_shape=jax.ShapeDtypeStruct((B, N, K_NBR), jnp.int32),
    )(xf, xf)


def _make_sc_gather(R, C):
    nc, ns = _sc_workers()
    rpw = R // (nc * ns)
    chunk = 128
    chunks = rpw // chunk
    mesh = plsc.VectorSubcoreMesh(core_axis_name="c", subcore_axis_name="s")

    @functools.partial(
        pl.kernel, mesh=mesh,
        out_type=jax.ShapeDtypeStruct((R, C), jnp.float32),
        scratch_types=[
            pltpu.VMEM((rpw,), jnp.int32),
            pltpu.VMEM((chunk, C), jnp.float32),
            pltpu.SemaphoreType.DMA,
        ],
    )
    def gath(idx_hbm, table_hbm, out_hbm, idx_v, rows_v, sem):
        wid = lax.axis_index("s") * nc + lax.axis_index("c")
        pltpu.sync_copy(idx_hbm.at[pl.ds(wid * rpw, rpw)], idx_v)

        def body(j, carry):
            pltpu.async_copy(
                table_hbm.at[idx_v.at[pl.ds(j * chunk, chunk)]],
                rows_v, sem).wait()
            pltpu.sync_copy(
                rows_v, out_hbm.at[pl.ds(wid * rpw + j * chunk, chunk)])
            return carry

        lax.fori_loop(0, chunks, body, 0)

    return gath


def _mm_body(g_ref, w_ref, b_ref, o_ref, *, relu):
    out = lax.dot_general(g_ref[0], w_ref[:], (((1,), (0,)), ((), ())),
                          preferred_element_type=jnp.float32)
    out = out + b_ref[:]
    if relu:
        out = jnp.maximum(out, 0.0)
    o_ref[0] = out


def _mm(G, wr, b, relu):
    B, N, KC = G.shape
    O = wr.shape[1]
    return pl.pallas_call(
        functools.partial(_mm_body, relu=relu),
        grid=(B, N // RB),
        in_specs=[
            pl.BlockSpec((1, RB, KC), lambda b, i: (b, i, 0)),
            pl.BlockSpec((KC, O), lambda b, i: (0, 0)),
            pl.BlockSpec((1, O), lambda b, i: (0, 0)),
        ],
        out_specs=pl.BlockSpec((1, RB, O), lambda b, i: (b, i, 0)),
        out_shape=jax.ShapeDtypeStruct((B, N, O), jnp.float32),
    )(G, wr, b.reshape(1, O))


def _conv_nn_tokens(xf, W, b, relu):
    B, N, C = xf.shape
    O, Cw, _ = W.shape
    idx = _topk(xf)
    table = xf.reshape(B * N, C)
    G = _make_sc_gather(B * N * K_NBR, C)(idx.reshape(-1), table)
    G = G.reshape(B, N, K_NBR * C)
    wr = jnp.pad(W.transpose(2, 1, 0), ((0, 0), (0, C - Cw), (0, 0)))
    return _mm(G, wr.reshape(K_NBR * C, O), b, relu)


def kernel(x, W1, b1, W2, b2, W3, b3):
    B = x.shape[0]
    h = _conv1(x, W1, b1)
    hu = h.reshape(B, 16, 64, 2, 64, 2).transpose(0, 1, 3, 5, 2, 4)
    xf2 = hu.reshape(B, 64, N_TOK).transpose(0, 2, 1)
    xf2 = jnp.pad(xf2, ((0, 0), (0, 0), (0, 64)))
    t3 = _conv_nn_tokens(xf2, W2, b2, relu=True)
    out = _conv_nn_tokens(t3, W3, b3, relu=False)
    out = out.transpose(0, 2, 1).reshape(B, 12, 64, 64)
    return _pixel_shuffle(out, 2)


# speedup vs baseline: 1.1680x; 1.1680x over previous
import functools

import jax
import jax.numpy as jnp
from jax import lax
from jax.experimental import pallas as pl
from jax.experimental.pallas import tpu as pltpu
from jax.experimental.pallas import tpu_sc as plsc

NEG = -3.0e38
N_TOK = 4096
K_NBR = 9
RB = 256

def _sc_workers():
    info = plsc.get_sparse_core_info()
    return info.num_cores, info.num_subcores


def _pixel_shuffle(x, r):
    B, C, H, W = x.shape
    x = x.reshape(B, C // (r * r), r, r, H, W)
    x = x.transpose(0, 1, 4, 2, 5, 3)
    return x.reshape(B, C // (r * r), H * r, W * r)


def _conv1_body(xp_ref, w_ref, b_ref, o_ref):
    acc = jnp.zeros((16, 128 * 128), jnp.float32)
    for dy in range(3):
        for dx in range(3):
            xs = xp_ref[0, :, dy:dy + 128, dx:dx + 128].reshape(3, 128 * 128)
            acc = acc + lax.dot_general(
                w_ref[:, :, dy, dx], xs, (((1,), (0,)), ((), ())),
                preferred_element_type=jnp.float32)
    acc = jnp.maximum(acc + b_ref[:], 0.0)
    o_ref[0] = acc.reshape(16, 128, 128)


def _conv1(x, W1, b1):
    B = x.shape[0]
    xp = jnp.pad(x, ((0, 0), (0, 0), (1, 1), (1, 1)))
    return pl.pallas_call(
        _conv1_body,
        grid=(B,),
        in_specs=[
            pl.BlockSpec((1, 3, 130, 130), lambda b: (b, 0, 0, 0)),
            pl.BlockSpec((16, 3, 3, 3), lambda b: (0, 0, 0, 0)),
            pl.BlockSpec((16, 1), lambda b: (0, 0)),
        ],
        out_specs=pl.BlockSpec((1, 16, 128, 128), lambda b: (b, 0, 0, 0)),
        out_shape=jax.ShapeDtypeStruct((B, 16, 128, 128), jnp.float32),
    )(xp, W1, b1.reshape(16, 1))


def _norm_body(x_ref, o_ref):
    x = x_ref[0]
    nrm = jnp.sqrt(jnp.sum(x * x, axis=1, keepdims=True))
    o_ref[0] = x * (1.0 / (nrm + 1e-8))


def _normalize(xf):
    B, N, C = xf.shape
    return pl.pallas_call(
        _norm_body,
        grid=(B,),
        in_specs=[pl.BlockSpec((1, N, C), lambda b: (b, 0, 0))],
        out_specs=pl.BlockSpec((1, N, C), lambda b: (b, 0, 0)),
        out_shape=jax.ShapeDtypeStruct((B, N, C), jnp.float32),
    )(xf)


def _topk_body(xn_ref, xr_ref, o_ref):
    xn = xn_ref[0]
    N = xn.shape[0]
    sim = lax.dot_general(xr_ref[0], xn, (((1,), (1,)), ((), ())),
                          preferred_element_type=jnp.float32)
    nblk = N // 128
    iota = lax.broadcasted_iota(jnp.int32, (RB, 128), 1)
    giota = lax.broadcasted_iota(jnp.int32, (RB, N), 1)
    base = pl.program_id(0) * N
    for k in range(K_NBR):
        best = jnp.full((RB, 128), NEG, jnp.float32)
        bblk = jnp.zeros((RB, 128), jnp.int32)
        for j in range(nblk):
            v = sim[:, j * 128:(j + 1) * 128]
            better = v > best
            best = jnp.maximum(best, v)
            bblk = jnp.where(better, j, bblk)
        mx = jnp.max(best, axis=1, keepdims=True)
        cand = jnp.where(best == mx, bblk * 128 + iota, N)
        idxk = jnp.min(cand, axis=1, keepdims=True)
        o_ref[0, :, k] = (idxk + base)[:, 0]
        sim = jnp.where(giota == idxk, NEG, sim)


def _topk(xn):
    B, N, C = xn.shape
    return pl.pallas_call(
        _topk_body,
        grid=(B, N // RB),
        in_specs=[
            pl.BlockSpec((1, N, C), lambda b, i: (b, 0, 0)),
            pl.BlockSpec((1, RB, C), lambda b, i: (b, i, 0)),
        ],
        out_specs=pl.BlockSpec((1, RB, K_NBR), lambda b, i: (b, i, 0)),
        out_shape=jax.ShapeDtypeStruct((B, N, K_NBR), jnp.int32),
    )(xn, xn)


def _make_sc_gather(R, C):
    nc, ns = _sc_workers()
    rpw = R // (nc * ns)
    chunk = 128
    chunks = rpw // chunk
    mesh = plsc.VectorSubcoreMesh(core_axis_name="c", subcore_axis_name="s")

    @functools.partial(
        pl.kernel, mesh=mesh,
        out_type=jax.ShapeDtypeStruct((R, C), jnp.float32),
        scratch_types=[
            pltpu.VMEM((rpw,), jnp.int32),
            pltpu.VMEM((chunk, C), jnp.float32),
            pltpu.SemaphoreType.DMA,
        ],
    )
    def gath(idx_hbm, table_hbm, out_hbm, idx_v, rows_v, sem):
        wid = lax.axis_index("s") * nc + lax.axis_index("c")
        pltpu.sync_copy(idx_hbm.at[pl.ds(wid * rpw, rpw)], idx_v)

        def body(j, carry):
            pltpu.async_copy(
                table_hbm.at[idx_v.at[pl.ds(j * chunk, chunk)]],
                rows_v, sem).wait()
            pltpu.sync_copy(
                rows_v, out_hbm.at[pl.ds(wid * rpw + j * chunk, chunk)])
            return carry

        lax.fori_loop(0, chunks, body, 0)

    return gath


def _mm_body(g_ref, w_ref, b_ref, o_ref, *, relu):
    out = lax.dot_general(g_ref[0], w_ref[:], (((1,), (0,)), ((), ())),
                          preferred_element_type=jnp.float32)
    out = out + b_ref[:]
    if relu:
        out = jnp.maximum(out, 0.0)
    o_ref[0] = out


def _mm(G, wr, b, relu):
    B, N, KC = G.shape
    O = wr.shape[1]
    return pl.pallas_call(
        functools.partial(_mm_body, relu=relu),
        grid=(B, N // RB),
        in_specs=[
            pl.BlockSpec((1, RB, KC), lambda b, i: (b, i, 0)),
            pl.BlockSpec((KC, O), lambda b, i: (0, 0)),
            pl.BlockSpec((1, O), lambda b, i: (0, 0)),
        ],
        out_specs=pl.BlockSpec((1, RB, O), lambda b, i: (b, i, 0)),
        out_shape=jax.ShapeDtypeStruct((B, N, O), jnp.float32),
    )(G, wr, b.reshape(1, O))


def _conv_nn_tokens(xf, W, b, relu):
    B, N, C = xf.shape
    O = W.shape[0]
    idx = _topk(_normalize(xf))
    Cp = max(C, 128)
    table = jnp.pad(xf, ((0, 0), (0, 0), (0, Cp - C))).reshape(B * N, Cp)
    G = _make_sc_gather(B * N * K_NBR, Cp)(idx.reshape(-1), table)
    G = G.reshape(B, N, K_NBR * Cp)
    wr = jnp.pad(W.transpose(2, 1, 0), ((0, 0), (0, Cp - C), (0, 0)))
    return _mm(G, wr.reshape(K_NBR * Cp, O), b, relu)


def kernel(x, W1, b1, W2, b2, W3, b3):
    B = x.shape[0]
    h = _conv1(x, W1, b1)
    hu = h.reshape(B, 16, 64, 2, 64, 2).transpose(0, 1, 3, 5, 2, 4)
    xf2 = hu.reshape(B, 64, N_TOK).transpose(0, 2, 1)
    t3 = _conv_nn_tokens(xf2, W2, b2, relu=True)
    out = _conv_nn_tokens(t3, W3, b3, relu=False)
    out = out.transpose(0, 2, 1).reshape(B, 12, 64, 64)
    return _pixel_shuffle(out, 2)


# speedup vs baseline: 1.2445x; 1.0655x over previous
import functools

import jax
import jax.numpy as jnp
from jax import lax
from jax.experimental import pallas as pl
from jax.experimental.pallas import tpu as pltpu
from jax.experimental.pallas import tpu_sc as plsc

NEG = -3.0e38
N_TOK = 4096
K_NBR = 9
RB = 256

def _sc_workers():
    info = plsc.get_sparse_core_info()
    return info.num_cores, info.num_subcores


def _pixel_shuffle(x, r):
    B, C, H, W = x.shape
    x = x.reshape(B, C // (r * r), r, r, H, W)
    x = x.transpose(0, 1, 4, 2, 5, 3)
    return x.reshape(B, C // (r * r), H * r, W * r)


def _conv1_body(xp_ref, w_ref, b_ref, o_ref):
    acc = jnp.zeros((16, 128 * 128), jnp.float32)
    for dy in range(3):
        for dx in range(3):
            xs = xp_ref[0, :, dy:dy + 128, dx:dx + 128].reshape(3, 128 * 128)
            acc = acc + lax.dot_general(
                w_ref[:, :, dy, dx], xs, (((1,), (0,)), ((), ())),
                preferred_element_type=jnp.float32)
    acc = jnp.maximum(acc + b_ref[:], 0.0)
    o_ref[0] = acc.reshape(16, 128, 128)


def _conv1(x, W1, b1):
    B = x.shape[0]
    xp = jnp.pad(x, ((0, 0), (0, 0), (1, 1), (1, 1)))
    return pl.pallas_call(
        _conv1_body,
        grid=(B,),
        in_specs=[
            pl.BlockSpec((1, 3, 130, 130), lambda b: (b, 0, 0, 0)),
            pl.BlockSpec((16, 3, 3, 3), lambda b: (0, 0, 0, 0)),
            pl.BlockSpec((16, 1), lambda b: (0, 0)),
        ],
        out_specs=pl.BlockSpec((1, 16, 128, 128), lambda b: (b, 0, 0, 0)),
        out_shape=jax.ShapeDtypeStruct((B, 16, 128, 128), jnp.float32),
    )(xp, W1, b1.reshape(16, 1))


def _norm_body(x_ref, o_ref):
    x = x_ref[0]
    nrm = jnp.sqrt(jnp.sum(x * x, axis=1, keepdims=True))
    o_ref[0] = x * (1.0 / (nrm + 1e-8))


def _normalize(xf):
    B, N, C = xf.shape
    return pl.pallas_call(
        _norm_body,
        grid=(B,),
        in_specs=[pl.BlockSpec((1, N, C), lambda b: (b, 0, 0))],
        out_specs=pl.BlockSpec((1, N, C), lambda b: (b, 0, 0)),
        out_shape=jax.ShapeDtypeStruct((B, N, C), jnp.float32),
    )(xf)


def _topk_body(xn_ref, xr_ref, o_ref):
    xn = xn_ref[0]
    N = xn.shape[0]
    sim = lax.dot_general(xr_ref[0], xn, (((1,), (1,)), ((), ())),
                          preferred_element_type=jnp.float32)
    nblk = N // 128
    iota = lax.broadcasted_iota(jnp.int32, (RB, 128), 1)
    giota = lax.broadcasted_iota(jnp.int32, (RB, N), 1)
    base = pl.program_id(0) * N
    for k in range(K_NBR):
        best = jnp.full((RB, 128), NEG, jnp.float32)
        bblk = jnp.zeros((RB, 128), jnp.int32)
        for j in range(nblk):
            v = sim[:, j * 128:(j + 1) * 128]
            better = v > best
            best = jnp.maximum(best, v)
            bblk = jnp.where(better, j, bblk)
        mx = jnp.max(best, axis=1, keepdims=True)
        cand = jnp.where(best == mx, bblk * 128 + iota, N)
        idxk = jnp.min(cand, axis=1, keepdims=True)
        o_ref[0, :, k] = (idxk + base)[:, 0]
        sim = jnp.where(giota == idxk, NEG, sim)


def _topk(xn, half):
    B, N, C = xn.shape
    nb = N // (2 * RB)
    off = half * nb
    return pl.pallas_call(
        _topk_body,
        grid=(B, nb),
        in_specs=[
            pl.BlockSpec((1, N, C), lambda b, i: (b, 0, 0)),
            pl.BlockSpec((1, RB, C), lambda b, i: (b, i + off, 0)),
        ],
        out_specs=pl.BlockSpec((1, RB, K_NBR), lambda b, i: (b, i, 0)),
        out_shape=jax.ShapeDtypeStruct((B, N // 2, K_NBR), jnp.int32),
    )(xn, xn)


def _make_sc_gather(R, C):
    nc, ns = _sc_workers()
    rpw = R // (nc * ns)
    chunk = 128
    chunks = rpw // chunk
    mesh = plsc.VectorSubcoreMesh(core_axis_name="c", subcore_axis_name="s")

    @functools.partial(
        pl.kernel, mesh=mesh,
        out_type=jax.ShapeDtypeStruct((R, C), jnp.float32),
        scratch_types=[
            pltpu.VMEM((rpw,), jnp.int32),
            pltpu.VMEM((chunk, C), jnp.float32),
            pltpu.SemaphoreType.DMA,
        ],
    )
    def gath(idx_hbm, table_hbm, out_hbm, idx_v, rows_v, sem):
        wid = lax.axis_index("s") * nc + lax.axis_index("c")
        pltpu.sync_copy(idx_hbm.at[pl.ds(wid * rpw, rpw)], idx_v)

        def body(j, carry):
            pltpu.async_copy(
                table_hbm.at[idx_v.at[pl.ds(j * chunk, chunk)]],
                rows_v, sem).wait()
            pltpu.sync_copy(
                rows_v, out_hbm.at[pl.ds(wid * rpw + j * chunk, chunk)])
            return carry

        lax.fori_loop(0, chunks, body, 0)

    return gath


def _mm_body(g_ref, w_ref, b_ref, o_ref, *, relu):
    out = lax.dot_general(g_ref[0], w_ref[:], (((1,), (0,)), ((), ())),
                          preferred_element_type=jnp.float32)
    out = out + b_ref[:]
    if relu:
        out = jnp.maximum(out, 0.0)
    o_ref[0] = out


def _mm(G, wr, b, relu):
    B, N, KC = G.shape
    O = wr.shape[1]
    return pl.pallas_call(
        functools.partial(_mm_body, relu=relu),
        grid=(B, N // RB),
        in_specs=[
            pl.BlockSpec((1, RB, KC), lambda b, i: (b, i, 0)),
            pl.BlockSpec((KC, O), lambda b, i: (0, 0)),
            pl.BlockSpec((1, O), lambda b, i: (0, 0)),
        ],
        out_specs=pl.BlockSpec((1, RB, O), lambda b, i: (b, i, 0)),
        out_shape=jax.ShapeDtypeStruct((B, N, O), jnp.float32),
    )(G, wr, b.reshape(1, O))


def _conv_nn_tokens(xf, W, b, relu):
    B, N, C = xf.shape
    O = W.shape[0]
    xn = _normalize(xf)
    Cp = max(C, 128)
    table = jnp.pad(xf, ((0, 0), (0, 0), (0, Cp - C))).reshape(B * N, Cp)
    wr = jnp.pad(W.transpose(2, 1, 0), ((0, 0), (0, Cp - C), (0, 0)))
    wr = wr.reshape(K_NBR * Cp, O)
    gath = _make_sc_gather(B * N * K_NBR // 2, Cp)
    idx0 = _topk(xn, 0)
    idx1 = _topk(xn, 1)
    outs = []
    for idx in (idx0, idx1):
        G = gath(idx.reshape(-1), table).reshape(B, N // 2, K_NBR * Cp)
        outs.append(_mm(G, wr, b, relu))
    return jnp.concatenate(outs, axis=1)


def kernel(x, W1, b1, W2, b2, W3, b3):
    B = x.shape[0]
    h = _conv1(x, W1, b1)
    hu = h.reshape(B, 16, 64, 2, 64, 2).transpose(0, 1, 3, 5, 2, 4)
    xf2 = hu.reshape(B, 64, N_TOK).transpose(0, 2, 1)
    t3 = _conv_nn_tokens(xf2, W2, b2, relu=True)
    out = _conv_nn_tokens(t3, W3, b3, relu=False)
    out = out.transpose(0, 2, 1).reshape(B, 12, 64, 64)
    return _pixel_shuffle(out, 2)
